# baseline (speedup 1.0000x reference)
import jax
import jax.numpy as jnp
from jax.experimental import pallas as pl
from jax.experimental.pallas import tpu as pltpu

B, D, K, CB = 4096, 4096, 256, 8192
BLK = 256
NBLK = B // BLK
CBC = 2048
NC = CB // CBC
DC = 1024
ND = D // DC
PG = 1024
NP = CB // PG


def _vq_kernel(x_ref, u_ref, cb_ref, hcn_ref,
               xr_ref, z_ref, zq_ref, idx_ref,
               ub_ref, cbb_ref):
    @pl.when(pl.program_id(0) == 0)
    def _prep():
        ub_ref[...] = u_ref[...].astype(jnp.bfloat16)
        cbb_ref[...] = cb_ref[...].astype(jnp.bfloat16)

    zparts = []
    for p in range(ND):
        xbp = x_ref[:, p * DC:(p + 1) * DC].astype(jnp.bfloat16)
        zparts.append(jax.lax.dot_general(
            xbp, ub_ref[p * DC:(p + 1) * DC, :],
            (((1,), (0,)), ((), ())),
            preferred_element_type=jnp.float32))
    while len(zparts) > 1:
        zparts = [zparts[i] + zparts[i + 1] for i in range(0, len(zparts), 2)]
    z = zparts[0]
    z_ref[...] = z
    zb = z.astype(jnp.bfloat16)

    big = jnp.int32(2**31 - 1)
    iota = jax.lax.broadcasted_iota(jnp.int32, (BLK, CBC), 1)
    ms, idxs = [], []
    for c in range(NC):
        sc = jax.lax.dot_general(zb, cbb_ref[c * CBC:(c + 1) * CBC, :],
                                 (((1,), (1,)), ((), ())),
                                 preferred_element_type=jnp.float32)
        d2c = hcn_ref[:, c * CBC:(c + 1) * CBC] - sc
        cm = jnp.min(d2c, axis=1, keepdims=True)
        ci = jnp.min(jnp.where(d2c == cm, iota, big), axis=1, keepdims=True)
        ms.append(cm)
        idxs.append(ci)

    m = ms[0]
    for c in range(1, NC):
        m = jnp.minimum(m, ms[c])
    idx = None
    for c in range(NC):
        cand = jnp.where(ms[c] == m, idxs[c] + c * CBC, big)
        idx = cand if idx is None else jnp.minimum(idx, cand)
    idx_ref[...] = idx

    lo = jax.lax.rem(idx, jnp.int32(PG))
    hi = jax.lax.div(idx, jnp.int32(PG))
    iota_lo = jax.lax.broadcasted_iota(jnp.int32, (BLK, PG), 1)
    onehot = jnp.where(iota_lo == lo, jnp.float32(1), jnp.float32(0)
                       ).astype(jnp.bfloat16)
    zq = None
    for p in range(NP):
        pc = jax.lax.dot_general(onehot, cbb_ref[p * PG:(p + 1) * PG, :],
                                 (((1,), (0,)), ((), ())),
                                 preferred_element_type=jnp.float32)
        sel = jnp.where(hi == p, pc, jnp.float32(0))
        zq = sel if zq is None else zq + sel
    zq_ref[...] = zq
    xr_ref[...] = jax.lax.dot_general(zq.astype(jnp.bfloat16), ub_ref[...],
                                      (((1,), (1,)), ((), ())),
                                      preferred_element_type=jnp.float32)


def kernel(x, U_k, codebook):
    hcn = (0.5 * jnp.sum(codebook * codebook, axis=1))[None, :]
    x_recon, z, z_q, idx = pl.pallas_call(
        _vq_kernel,
        grid=(NBLK,),
        in_specs=[
            pl.BlockSpec((BLK, D), lambda i: (i, 0)),
            pl.BlockSpec((D, K), lambda i: (0, 0)),
            pl.BlockSpec((CB, K), lambda i: (0, 0)),
            pl.BlockSpec((1, CB), lambda i: (0, 0)),
        ],
        out_specs=[
            pl.BlockSpec((BLK, D), lambda i: (i, 0)),
            pl.BlockSpec((BLK, K), lambda i: (i, 0)),
            pl.BlockSpec((BLK, K), lambda i: (i, 0)),
            pl.BlockSpec((BLK, 1), lambda i: (i, 0)),
        ],
        out_shape=[
            jax.ShapeDtypeStruct((B, D), jnp.float32),
            jax.ShapeDtypeStruct((B, K), jnp.float32),
            jax.ShapeDtypeStruct((B, K), jnp.float32),
            jax.ShapeDtypeStruct((B, 1), jnp.int32),
        ],
        scratch_shapes=[
            pltpu.VMEM((D, K), jnp.bfloat16),
            pltpu.VMEM((CB, K), jnp.bfloat16),
        ],
    )(x, U_k, codebook, hcn)
    return (x_recon, z, z_q, idx.reshape(B))


# speedup vs baseline: 1.0360x; 1.0360x over previous
import jax
import jax.numpy as jnp
from jax.experimental import pallas as pl
from jax.experimental.pallas import tpu as pltpu

B, D, K, CB = 4096, 4096, 256, 8192
BLK = 256
NBLK = B // BLK
CBC = 2048
NC = CB // CBC
DC = 1024
ND = D // DC
PG = 1024
NP = CB // PG


def _vq_kernel(x_ref, u_ref, cb_ref, hcn_ref,
               xr_ref, z_ref, zq_ref, idx_ref,
               ub_ref, cbb_ref, zs_ref, ixs_ref):
    @pl.when(pl.program_id(0) == 0)
    def _prep():
        ub_ref[...] = u_ref[...].astype(jnp.bfloat16)
        cbb_ref[...] = cb_ref[...].astype(jnp.bfloat16)

    zp = zs_ref[...]
    idxp = ixs_ref[...]
    lo = jax.lax.rem(idxp, jnp.int32(PG))
    hi = jax.lax.div(idxp, jnp.int32(PG))
    iota_lo = jax.lax.broadcasted_iota(jnp.int32, (BLK, PG), 1)
    onehot = jnp.where(iota_lo == lo, jnp.float32(1), jnp.float32(0)
                       ).astype(jnp.bfloat16)

    zparts = []
    for p in range(ND):
        xbp = x_ref[:, p * DC:(p + 1) * DC].astype(jnp.bfloat16)
        zparts.append(jax.lax.dot_general(
            xbp, ub_ref[p * DC:(p + 1) * DC, :],
            (((1,), (0,)), ((), ())),
            preferred_element_type=jnp.float32))
    while len(zparts) > 1:
        zparts = [zparts[i] + zparts[i + 1] for i in range(0, len(zparts), 2)]
    z = zparts[0]
    zb = z.astype(jnp.bfloat16)

    big = jnp.int32(2**31 - 1)
    iota = jax.lax.broadcasted_iota(jnp.int32, (BLK, CBC), 1)
    ms, idxs = [], []
    zq = None
    ppg = 2 * NP // NC
    for c in range(NC):
        sc = jax.lax.dot_general(zb, cbb_ref[c * CBC:(c + 1) * CBC, :],
                                 (((1,), (1,)), ((), ())),
                                 preferred_element_type=jnp.float32)
        if c < NC // 2:
            for p in range(c * ppg, (c + 1) * ppg):
                pc = jax.lax.dot_general(onehot, cbb_ref[p * PG:(p + 1) * PG, :],
                                         (((1,), (0,)), ((), ())),
                                         preferred_element_type=jnp.float32)
                sel = jnp.where(hi == p, pc, jnp.float32(0))
                zq = sel if zq is None else zq + sel
        elif c == NC // 2:
            zq_ref[...] = zq
            z_ref[...] = zp
            idx_ref[...] = idxp
            xr_ref[...] = jax.lax.dot_general(
                zq.astype(jnp.bfloat16), ub_ref[...],
                (((1,), (1,)), ((), ())),
                preferred_element_type=jnp.float32)
        d2c = hcn_ref[:, c * CBC:(c + 1) * CBC] - sc
        cm = jnp.min(d2c, axis=1, keepdims=True)
        ci = jnp.min(jnp.where(d2c == cm, iota, big), axis=1, keepdims=True)
        ms.append(cm)
        idxs.append(ci)

    m = ms[0]
    for c in range(1, NC):
        m = jnp.minimum(m, ms[c])
    idx = None
    for c in range(NC):
        cand = jnp.where(ms[c] == m, idxs[c] + c * CBC, big)
        idx = cand if idx is None else jnp.minimum(idx, cand)

    zs_ref[...] = z
    ixs_ref[...] = idx


def kernel(x, U_k, codebook):
    hcn = (0.5 * jnp.sum(codebook * codebook, axis=1))[None, :]
    x_recon, z, z_q, idx = pl.pallas_call(
        _vq_kernel,
        grid=(NBLK + 1,),
        in_specs=[
            pl.BlockSpec((BLK, D), lambda i: (jnp.minimum(i, NBLK - 1), 0)),
            pl.BlockSpec((D, K), lambda i: (0, 0)),
            pl.BlockSpec((CB, K), lambda i: (0, 0)),
            pl.BlockSpec((1, CB), lambda i: (0, 0)),
        ],
        out_specs=[
            pl.BlockSpec((BLK, D), lambda i: (jnp.maximum(i - 1, 0), 0)),
            pl.BlockSpec((BLK, K), lambda i: (jnp.maximum(i - 1, 0), 0)),
            pl.BlockSpec((BLK, K), lambda i: (jnp.maximum(i - 1, 0), 0)),
            pl.BlockSpec((BLK, 1), lambda i: (jnp.maximum(i - 1, 0), 0)),
        ],
        out_shape=[
            jax.ShapeDtypeStruct((B, D), jnp.float32),
            jax.ShapeDtypeStruct((B, K), jnp.float32),
            jax.ShapeDtypeStruct((B, K), jnp.float32),
            jax.ShapeDtypeStruct((B, 1), jnp.int32),
        ],
        scratch_shapes=[
            pltpu.VMEM((D, K), jnp.bfloat16),
            pltpu.VMEM((CB, K), jnp.bfloat16),
            pltpu.VMEM((BLK, K), jnp.float32),
            pltpu.VMEM((BLK, 1), jnp.int32),
        ],
    )(x, U_k, codebook, hcn)
    return (x_recon, z, z_q, idx.reshape(B))
